# Initial kernel scaffold; baseline (speedup 1.0000x reference)
import jax
import jax.numpy as jnp
from jax.experimental import pallas as pl


def kernel(coords, image):
    raise NotImplementedError("write your pallas kernel here")


# speedup vs baseline: 23.8157x; 23.8157x over previous
import jax
import jax.numpy as jnp
from jax import lax
from jax.experimental import pallas as pl
from jax.experimental.pallas import tpu as pltpu
from jax.experimental.pallas import tpu_sc as plsc

_W = 512
NC, NS, L = 2, 16, 16
NW = NC * NS
B = 1048576
BPW = B // NW
CHUNK = 2048
NIDX = 128
D = 8


def _body(coords_hbm, table_hbm, out_hbm, coords_v, idx_v, rows_v, sem):
    wid = lax.axis_index("s") * NC + lax.axis_index("c")
    base = wid * BPW
    lane2 = lax.iota(jnp.int32, L) * 2

    def chunk_body(t, carry):
        off = base + t * CHUNK
        pltpu.sync_copy(coords_hbm.at[pl.ds(2 * off, 2 * CHUNK)], coords_v)

        def idx_body(k, c2):
            for jj in range(NIDX // L):
                pos = k * (2 * NIDX) + jj * (2 * L) + lane2
                r = plsc.load_gather(coords_v, [pos])
                c = plsc.load_gather(coords_v, [pos + 1])
                idx_v[k, pl.ds(jj * L, L)] = r * _W + c
            return c2

        lax.fori_loop(0, CHUNK // NIDX, idx_body, 0)

        def g_body(k, c2):
            pltpu.async_copy(
                table_hbm.at[idx_v.at[k]],
                rows_v.at[pl.ds(k * NIDX, NIDX)],
                sem,
            ).wait()
            return c2

        lax.fori_loop(0, CHUNK // NIDX, g_body, 0)
        pltpu.sync_copy(rows_v, out_hbm.at[pl.ds(off, CHUNK)])
        return carry

    lax.fori_loop(0, BPW // CHUNK, chunk_body, 0)


def kernel(coords, image):
    coords_flat = coords.reshape(-1).astype(jnp.int32)
    table = jnp.pad(image.reshape(_W * _W, 3), ((0, 0), (0, D - 3)))
    f = pl.kernel(
        _body,
        out_type=jax.ShapeDtypeStruct((B, D), jnp.float32),
        mesh=plsc.VectorSubcoreMesh(core_axis_name="c", subcore_axis_name="s"),
        compiler_params=pltpu.CompilerParams(
            needs_layout_passes=False, use_tc_tiling_on_sc=False
        ),
        scratch_types=[
            pltpu.VMEM((2 * CHUNK,), jnp.int32),
            pltpu.VMEM((CHUNK // NIDX, NIDX), jnp.int32),
            pltpu.VMEM((CHUNK, D), jnp.float32),
            pltpu.SemaphoreType.DMA,
        ],
    )
    return f(coords_flat, table)[:, :3]


# speedup vs baseline: 25.6222x; 1.0759x over previous
import jax
import jax.numpy as jnp
from jax import lax
from jax.experimental import pallas as pl
from jax.experimental.pallas import tpu as pltpu
from jax.experimental.pallas import tpu_sc as plsc

_W = 512
NC, NS, L = 2, 16, 16
NW = NC * NS
B = 1048576
BPW = B // NW
CHUNK = 2048
NIDX = 128
NB = CHUNK // NIDX
NCH = BPW // CHUNK
D = 8


def _coords_copy(coords_hbm, coords_v, sem_c, t, base, slot):
    off = base + t * CHUNK
    return pltpu.make_async_copy(
        coords_hbm.at[pl.ds(2 * off, 2 * CHUNK)], coords_v.at[slot], sem_c.at[slot]
    )


def _body(coords_hbm, table_hbm, out_hbm, coords_v, idx_v, rows_v, sem_c, sem_g):
    wid = lax.axis_index("s") * NC + lax.axis_index("c")
    base = wid * BPW
    lane2 = lax.iota(jnp.int32, L) * 2

    _coords_copy(coords_hbm, coords_v, sem_c, 0, base, 0).start()

    def pair_body(tt, carry):
        for slot in range(2):
            t = 2 * tt + slot
            off = base + t * CHUNK
            _coords_copy(coords_hbm, coords_v, sem_c, t, base, slot).wait()

            @pl.when(t + 1 < NCH)
            def _():
                _coords_copy(
                    coords_hbm, coords_v, sem_c, t + 1, base, 1 - slot
                ).start()

            def idx_body(k, c2):
                for jj in range(NIDX // L):
                    pos = k * (2 * NIDX) + jj * (2 * L) + lane2
                    r = plsc.load_gather(coords_v.at[slot], [pos])
                    c = plsc.load_gather(coords_v.at[slot], [pos + 1])
                    idx_v[k, pl.ds(jj * L, L)] = r * _W + c
                return c2

            lax.fori_loop(0, NB, idx_body, 0)

            descs = [
                pltpu.async_copy(
                    table_hbm.at[idx_v.at[k]],
                    rows_v.at[pl.ds(k * NIDX, NIDX)],
                    sem_g,
                )
                for k in range(NB)
            ]
            for d in descs:
                d.wait()
            pltpu.sync_copy(rows_v, out_hbm.at[pl.ds(off, CHUNK)])
        return carry

    lax.fori_loop(0, NCH // 2, pair_body, 0)


def kernel(coords, image):
    coords_flat = coords.reshape(-1).astype(jnp.int32)
    table = jnp.pad(image.reshape(_W * _W, 3), ((0, 0), (0, D - 3)))
    f = pl.kernel(
        _body,
        out_type=jax.ShapeDtypeStruct((B, D), jnp.float32),
        mesh=plsc.VectorSubcoreMesh(core_axis_name="c", subcore_axis_name="s"),
        compiler_params=pltpu.CompilerParams(
            needs_layout_passes=False, use_tc_tiling_on_sc=False
        ),
        scratch_types=[
            pltpu.VMEM((2, 2 * CHUNK), jnp.int32),
            pltpu.VMEM((NB, NIDX), jnp.int32),
            pltpu.VMEM((CHUNK, D), jnp.float32),
            pltpu.SemaphoreType.DMA((2,)),
            pltpu.SemaphoreType.DMA,
        ],
    )
    return f(coords_flat, table)[:, :3]
